# baseline (speedup 1.0000x reference)
import jax
import jax.numpy as jnp
from jax.experimental import pallas as pl
from jax.experimental.pallas import tpu as pltpu

_CAP_ROWS = 262144
_ROW_DIM = 128
_NCHUNK = 8
_CHUNK = _CAP_ROWS // _NCHUNK


def _pushback_body(end_ref, data_ref, buf_ref, out_ref, sems, row_sem):
    copies = []
    for k in range(_NCHUNK):
        c = pltpu.make_async_copy(
            buf_ref.at[pl.ds(k * _CHUNK, _CHUNK), :],
            out_ref.at[pl.ds(k * _CHUNK, _CHUNK), :],
            sems.at[k],
        )
        c.start()
        copies.append(c)
    for c in copies:
        c.wait()
    row = pltpu.make_async_copy(
        data_ref,
        out_ref.at[pl.ds(end_ref[0], 1), :],
        row_sem,
    )
    row.start()
    row.wait()


def kernel(data, buffer, start_included, end_excluded, length):
    end = jnp.asarray(end_excluded, jnp.int32).reshape(1)
    data2 = data.reshape(1, _ROW_DIM)
    return pl.pallas_call(
        _pushback_body,
        in_specs=[
            pl.BlockSpec(memory_space=pltpu.SMEM),
            pl.BlockSpec(memory_space=pl.ANY),
            pl.BlockSpec(memory_space=pl.ANY),
        ],
        out_specs=pl.BlockSpec(memory_space=pl.ANY),
        out_shape=jax.ShapeDtypeStruct((_CAP_ROWS, _ROW_DIM), jnp.float32),
        scratch_shapes=[
            pltpu.SemaphoreType.DMA((_NCHUNK,)),
            pltpu.SemaphoreType.DMA,
        ],
    )(end, data2, buffer)


# speedup vs baseline: 48.7867x; 48.7867x over previous
import jax
import jax.numpy as jnp
from jax.experimental import pallas as pl
from jax.experimental.pallas import tpu as pltpu

_CAP_ROWS = 262144
_ROW_DIM = 128
_BLOCK = 16384


def _pushback_body(end_ref, data_ref, buf_ref, out_ref):
    out_ref[...] = buf_ref[...]
    i = pl.program_id(0)
    local = end_ref[0] - i * _BLOCK

    @pl.when((local >= 0) & (local < _BLOCK))
    def _():
        out_ref[pl.ds(local, 1), :] = data_ref[...]


def kernel(data, buffer, start_included, end_excluded, length):
    end = jnp.asarray(end_excluded, jnp.int32).reshape(1)
    data2 = data.reshape(1, _ROW_DIM)
    return pl.pallas_call(
        _pushback_body,
        grid=(_CAP_ROWS // _BLOCK,),
        in_specs=[
            pl.BlockSpec(memory_space=pltpu.SMEM),
            pl.BlockSpec((1, _ROW_DIM), lambda i: (0, 0)),
            pl.BlockSpec((_BLOCK, _ROW_DIM), lambda i: (i, 0)),
        ],
        out_specs=pl.BlockSpec((_BLOCK, _ROW_DIM), lambda i: (i, 0)),
        out_shape=jax.ShapeDtypeStruct((_CAP_ROWS, _ROW_DIM), jnp.float32),
        compiler_params=pltpu.CompilerParams(
            dimension_semantics=("arbitrary",),
        ),
    )(end, data2, buffer)
